# Initial kernel scaffold; baseline (speedup 1.0000x reference)
import jax
import jax.numpy as jnp
from jax.experimental import pallas as pl


def kernel(x, edge_index, Wq, Wk, Wv, Wo, ln_scale, ln_bias):
    raise NotImplementedError("write your pallas kernel here")


# speedup vs baseline: 6.6947x; 6.6947x over previous
import jax
import jax.numpy as jnp
from jax import lax
from jax.experimental import pallas as pl
from jax.experimental.pallas import tpu as pltpu
from jax.experimental.pallas import tpu_sc as plsc

N = 10000
E = 320000
D = 128
H = 8
DH = 16

NUM_CORES = 2
NUM_SUBCORES = 16
NUM_TILES = NUM_CORES * NUM_SUBCORES
B = 80
NBLK = E // B
BLKS_PER_TILE = NBLK // NUM_TILES
BLKS_PER_HALF = NBLK // 2
MB = H * DH * B
DHH = DH // 2


def _qkv_body(x_ref, wq_ref, wk_ref, wv_ref, q_ref, k_ref, v_ref):
    xb = x_ref[...]
    q_ref[...] = jnp.dot(xb, wq_ref[...], preferred_element_type=jnp.float32)
    k_ref[...] = jnp.dot(xb, wk_ref[...], preferred_element_type=jnp.float32)
    v_ref[...] = jnp.dot(xb, wv_ref[...], preferred_element_type=jnp.float32)


def _qkv_call(x, wq, wk, wv):
    bn = 1000
    w_spec = pl.BlockSpec((D, D), lambda i: (0, 0))
    r_spec = pl.BlockSpec((bn, D), lambda i: (i, 0))
    return pl.pallas_call(
        _qkv_body,
        grid=(N // bn,),
        in_specs=[r_spec, w_spec, w_spec, w_spec],
        out_specs=[r_spec, r_spec, r_spec],
        out_shape=[
            jax.ShapeDtypeStruct((N, D), jnp.float32),
            jax.ShapeDtypeStruct((N, D), jnp.float32),
            jax.ShapeDtypeStruct((N, D), jnp.float32),
        ],
    )(x, wq, wk, wv)


def _sc_msg_body(q_hbm, k_hbm, v_hbm, src_hbm, dst_hbm, msg_out, ex_out,
                 idx_src, idx_dst, q_buf, kv_buf, msg_buf, ex_buf):
    c = lax.axis_index("c")
    s = lax.axis_index("s")
    wid = s * NUM_CORES + c
    lane = jnp.arange(16, dtype=jnp.int32)
    zero16 = jnp.zeros((16,), jnp.float32)

    def _block(blk, _):
        gblk = wid * BLKS_PER_TILE + blk
        base = gblk * B
        pltpu.sync_copy(src_hbm.at[pl.ds(base, B)], idx_src)
        pltpu.sync_copy(dst_hbm.at[pl.ds(base, B)], idx_dst)
        pltpu.sync_copy(q_hbm.at[idx_dst], q_buf)
        pltpu.sync_copy(k_hbm.at[idx_src], kv_buf)

        def _score_group(g, _):
            rows = g * 16 + lane

            def _score_head(h, _):
                acc = zero16
                for d in range(DH):
                    col = jnp.full((16,), h * DH + d, jnp.int32)
                    qv = plsc.load_gather(q_buf, [rows, col])
                    kv = plsc.load_gather(kv_buf, [rows, col])
                    acc = acc + qv * kv
                ex = jnp.exp(acc * 0.25)
                plsc.store_scatter(ex_buf, [h * B + rows], ex)
                return 0

            return lax.fori_loop(0, H, _score_head, 0)

        lax.fori_loop(0, B // 16, _score_group, 0)

        pltpu.sync_copy(v_hbm.at[idx_src], kv_buf)

        def _msg_group(g, _):
            rows = g * 16 + lane

            def _msg_head(h, _):
                exv = plsc.load_gather(ex_buf, [h * B + rows])
                for d in range(DH):
                    colv = jnp.full((16,), h * DH + d, jnp.int32)
                    vv = plsc.load_gather(kv_buf, [rows, colv])
                    plsc.store_scatter(msg_buf,
                                       [(h * DH + d) * B + rows], vv * exv)
                return 0

            return lax.fori_loop(0, H, _msg_head, 0)

        lax.fori_loop(0, B // 16, _msg_group, 0)
        pltpu.sync_copy(msg_buf, msg_out.at[pl.ds(gblk * MB, MB)])
        pltpu.sync_copy(ex_buf, ex_out.at[pl.ds(gblk * H * B, H * B)])
        return 0

    lax.fori_loop(0, BLKS_PER_TILE, _block, 0)


def _sc_msg_call(q, k, v, src, dst):
    mesh = plsc.VectorSubcoreMesh(core_axis_name="c", subcore_axis_name="s")
    f = pl.kernel(
        _sc_msg_body,
        out_type=[
            jax.ShapeDtypeStruct((NBLK * MB,), jnp.float32),
            jax.ShapeDtypeStruct((NBLK * H * B,), jnp.float32),
        ],
        mesh=mesh,
        scratch_types=[
            pltpu.VMEM((B,), jnp.int32),
            pltpu.VMEM((B,), jnp.int32),
            pltpu.VMEM((B, D), jnp.float32),
            pltpu.VMEM((B, D), jnp.float32),
            pltpu.VMEM((MB,), jnp.float32),
            pltpu.VMEM((H * B,), jnp.float32),
        ],
        compiler_params=pltpu.CompilerParams(needs_layout_passes=False),
    )
    return f(q, k, v, src, dst)


def _sc_agg_body(dst_hbm, ex_hbm, msg_hbm, num_out, den_out,
                 idx_dst, ex_s, msg_s, num_acc, den_acc):
    c = lax.axis_index("c")
    s = lax.axis_index("s")
    wid = s * NUM_CORES + c
    ht = wid // 4
    dhh = (wid // 2) % 2
    eh = wid % 2
    lane = jnp.arange(16, dtype=jnp.int32)
    zero16 = jnp.zeros((16,), jnp.float32)
    den_mask = jnp.full((16,), dhh == 0)

    def _zero_num(i, _):
        plsc.store_scatter(num_acc, [i * 16 + lane], zero16)
        return 0

    lax.fori_loop(0, N * DHH // 16, _zero_num, 0)

    def _zero_den(i, _):
        plsc.store_scatter(den_acc, [i * 16 + lane], zero16)
        return 0

    lax.fori_loop(0, N // 16, _zero_den, 0)

    def _block(blk, _):
        gblk = eh * BLKS_PER_HALF + blk
        pltpu.sync_copy(dst_hbm.at[pl.ds(gblk * B, B)], idx_dst)
        pltpu.sync_copy(ex_hbm.at[pl.ds(gblk * H * B + ht * B, B)], ex_s)
        pltpu.sync_copy(
            msg_hbm.at[pl.ds(gblk * MB + (ht * DH + dhh * DHH) * B,
                             DHH * B)], msg_s)

        def _group(g, _):
            rows = g * 16 + lane
            dstv = plsc.load_gather(idx_dst, [rows])
            exv = plsc.load_gather(ex_s, [rows])
            plsc.addupdate_scatter(den_acc, [dstv], exv, mask=den_mask)
            for d in range(DHH):
                mv = plsc.load_gather(msg_s, [d * B + rows])
                plsc.addupdate_scatter(num_acc, [dstv * DHH + d], mv)
            return 0

        lax.fori_loop(0, B // 16, _group, 0)
        return 0

    lax.fori_loop(0, BLKS_PER_HALF, _block, 0)

    pltpu.sync_copy(num_acc, num_out.at[pl.ds(wid * N * DHH, N * DHH)])
    pltpu.sync_copy(den_acc, den_out.at[pl.ds(wid * N, N)])


def _sc_agg_call(dst, ex, msg):
    mesh = plsc.VectorSubcoreMesh(core_axis_name="c", subcore_axis_name="s")
    f = pl.kernel(
        _sc_agg_body,
        out_type=[
            jax.ShapeDtypeStruct((NUM_TILES * N * DHH,), jnp.float32),
            jax.ShapeDtypeStruct((NUM_TILES * N,), jnp.float32),
        ],
        mesh=mesh,
        scratch_types=[
            pltpu.VMEM((B,), jnp.int32),
            pltpu.VMEM((B,), jnp.float32),
            pltpu.VMEM((DHH * B,), jnp.float32),
            pltpu.VMEM((N * DHH,), jnp.float32),
            pltpu.VMEM((N,), jnp.float32),
        ],
        compiler_params=pltpu.CompilerParams(needs_layout_passes=False),
    )
    return f(dst, ex, msg)


def _final_body(*refs):
    num_refs = refs[0:32]
    den_refs = refs[32:48]
    x_ref, wo_ref, lns_ref, lnb_ref, out_ref = refs[48:]
    agg_cols = []
    den_cols = []
    for h in range(H):
        for dhh in range(2):
            i0 = h * 4 + dhh * 2
            agg_cols.append(num_refs[i0][0] + num_refs[i0 + 1][0])
        dsum = den_refs[2 * h][0] + den_refs[2 * h + 1][0]
        den_cols.append(jnp.broadcast_to(dsum, (dsum.shape[0], DH)))
    agg = jnp.concatenate(agg_cols, axis=1)
    dfull = jnp.concatenate(den_cols, axis=1)
    aggn = agg / (dfull + 1e-9)
    out = jnp.dot(aggn, wo_ref[...], preferred_element_type=jnp.float32)
    hres = x_ref[...] + out
    mu = jnp.mean(hres, axis=1, keepdims=True)
    var = jnp.mean((hres - mu) ** 2, axis=1, keepdims=True)
    out_ref[...] = ((hres - mu) / jnp.sqrt(var + 1e-5)) * lns_ref[...] \
        + lnb_ref[...]


def _final_call(num, den, x, wo, lns, lnb):
    bn = 1000

    def _num_spec(tile):
        return pl.BlockSpec((1, bn, DHH), lambda i, _t=tile: (_t, i, 0))

    def _den_spec(tile):
        return pl.BlockSpec((1, bn, 1), lambda i, _t=tile: (_t, i, 0))

    in_specs = [_num_spec(h * 4 + dhh * 2 + e)
                for h in range(H) for dhh in range(2) for e in range(2)]
    in_specs += [_den_spec(h * 4 + e) for h in range(H) for e in range(2)]
    in_specs += [
        pl.BlockSpec((bn, D), lambda i: (i, 0)),
        pl.BlockSpec((D, D), lambda i: (0, 0)),
        pl.BlockSpec((1, D), lambda i: (0, 0)),
        pl.BlockSpec((1, D), lambda i: (0, 0)),
    ]
    num3 = num.reshape(NUM_TILES, N, DHH)
    den3 = den.reshape(NUM_TILES, N, 1)
    args = [num3] * 32 + [den3] * 16 + [x, wo, lns, lnb]
    return pl.pallas_call(
        _final_body,
        grid=(N // bn,),
        in_specs=in_specs,
        out_specs=pl.BlockSpec((bn, D), lambda i: (i, 0)),
        out_shape=jax.ShapeDtypeStruct((N, D), jnp.float32),
    )(*args)


def kernel(x, edge_index, Wq, Wk, Wv, Wo, ln_scale, ln_bias):
    q, k, v = _qkv_call(x, Wq, Wk, Wv)
    src = edge_index[0]
    dst = edge_index[1]
    msg, ex = _sc_msg_call(q, k, v, src, dst)
    num, den = _sc_agg_call(dst, ex, msg)
    out = _final_call(num, den, x, Wo,
                      ln_scale.reshape(1, D), ln_bias.reshape(1, D))
    return out


# speedup vs baseline: 9.1152x; 1.3616x over previous
import jax
import jax.numpy as jnp
from jax import lax
from jax.experimental import pallas as pl
from jax.experimental.pallas import tpu as pltpu
from jax.experimental.pallas import tpu_sc as plsc

N = 10000
E = 320000
D = 128
H = 8
DH = 16

NUM_CORES = 2
NUM_SUBCORES = 16
NUM_TILES = NUM_CORES * NUM_SUBCORES
B = 80
NBLK = E // B
BLKS_PER_TILE = NBLK // NUM_TILES
BLKS_PER_HALF = NBLK // 2
MB = H * DH * B
DHH = DH // 2


def _qkv_body(x_ref, wq_ref, wk_ref, wv_ref, q_ref, k_ref, v_ref):
    xb = x_ref[...]
    q_ref[...] = jnp.dot(xb, wq_ref[...], preferred_element_type=jnp.float32)
    k_ref[...] = jnp.dot(xb, wk_ref[...], preferred_element_type=jnp.float32)
    v_ref[...] = jnp.dot(xb, wv_ref[...], preferred_element_type=jnp.float32)


def _qkv_call(x, wq, wk, wv):
    bn = 1000
    w_spec = pl.BlockSpec((D, D), lambda i: (0, 0))
    r_spec = pl.BlockSpec((bn, D), lambda i: (i, 0))
    return pl.pallas_call(
        _qkv_body,
        grid=(N // bn,),
        in_specs=[r_spec, w_spec, w_spec, w_spec],
        out_specs=[r_spec, r_spec, r_spec],
        out_shape=[
            jax.ShapeDtypeStruct((N, D), jnp.float32),
            jax.ShapeDtypeStruct((N, D), jnp.float32),
            jax.ShapeDtypeStruct((N, D), jnp.float32),
        ],
    )(x, wq, wk, wv)


def _sc_msg_body(q_hbm, k_hbm, v_hbm, src_hbm, dst_hbm, msg_out, ex_out,
                 idx_src, idx_dst, q_buf, kv_buf, msg_buf, ex_buf, sem):
    c = lax.axis_index("c")
    s = lax.axis_index("s")
    wid = s * NUM_CORES + c
    lane = jnp.arange(16, dtype=jnp.int32)
    zero16 = jnp.zeros((16,), jnp.float32)

    def _block(blk, _):
        gblk = wid * BLKS_PER_TILE + blk
        base = gblk * B
        d1 = pltpu.async_copy(src_hbm.at[pl.ds(base, B)], idx_src, sem)
        d2 = pltpu.async_copy(dst_hbm.at[pl.ds(base, B)], idx_dst, sem)
        d1.wait()
        d2.wait()
        d3 = pltpu.async_copy(q_hbm.at[idx_dst], q_buf, sem)
        d4 = pltpu.async_copy(k_hbm.at[idx_src], kv_buf, sem)
        d3.wait()
        d4.wait()

        def _score_group(g, _):
            rows = g * 16 + lane

            def _score_head(h, _):
                acc = zero16
                for d in range(DH):
                    col = jnp.full((16,), h * DH + d, jnp.int32)
                    qv = plsc.load_gather(q_buf, [rows, col])
                    kv = plsc.load_gather(kv_buf, [rows, col])
                    acc = acc + qv * kv
                ex = jnp.exp(acc * 0.25)
                plsc.store_scatter(ex_buf, [h * B + rows], ex)
                return 0

            return lax.fori_loop(0, H, _score_head, 0)

        lax.fori_loop(0, B // 16, _score_group, 0)

        pltpu.sync_copy(v_hbm.at[idx_src], kv_buf)

        def _msg_group(g, _):
            rows = g * 16 + lane

            def _msg_head(h, _):
                exv = plsc.load_gather(ex_buf, [h * B + rows])
                for d in range(DH):
                    colv = jnp.full((16,), h * DH + d, jnp.int32)
                    vv = plsc.load_gather(kv_buf, [rows, colv])
                    plsc.store_scatter(msg_buf,
                                       [(h * DH + d) * B + rows], vv * exv)
                return 0

            return lax.fori_loop(0, H, _msg_head, 0)

        lax.fori_loop(0, B // 16, _msg_group, 0)
        d5 = pltpu.async_copy(msg_buf, msg_out.at[pl.ds(gblk * MB, MB)], sem)
        d6 = pltpu.async_copy(ex_buf, ex_out.at[pl.ds(gblk * H * B, H * B)],
                              sem)
        d5.wait()
        d6.wait()
        return 0

    lax.fori_loop(0, BLKS_PER_TILE, _block, 0)


def _sc_msg_call(q, k, v, src, dst):
    mesh = plsc.VectorSubcoreMesh(core_axis_name="c", subcore_axis_name="s")
    f = pl.kernel(
        _sc_msg_body,
        out_type=[
            jax.ShapeDtypeStruct((NBLK * MB,), jnp.float32),
            jax.ShapeDtypeStruct((NBLK * H * B,), jnp.float32),
        ],
        mesh=mesh,
        scratch_types=[
            pltpu.VMEM((B,), jnp.int32),
            pltpu.VMEM((B,), jnp.int32),
            pltpu.VMEM((B, D), jnp.float32),
            pltpu.VMEM((B, D), jnp.float32),
            pltpu.VMEM((MB,), jnp.float32),
            pltpu.VMEM((H * B,), jnp.float32),
            pltpu.SemaphoreType.DMA,
        ],
        compiler_params=pltpu.CompilerParams(needs_layout_passes=False),
    )
    return f(q, k, v, src, dst)


def _sc_agg_body(dst_hbm, ex_hbm, msg_hbm, num_out, den_out,
                 idx_dst, ex_s, msg_s, num_acc, den_acc, sem):
    c = lax.axis_index("c")
    s = lax.axis_index("s")
    wid = s * NUM_CORES + c
    ht = wid // 4
    dhh = (wid // 2) % 2
    eh = wid % 2
    lane = jnp.arange(16, dtype=jnp.int32)
    zero16 = jnp.zeros((16,), jnp.float32)
    den_mask = jnp.full((16,), dhh == 0)

    def _zero_num(i, _):
        plsc.store_scatter(num_acc, [i * 16 + lane], zero16)
        return 0

    lax.fori_loop(0, N * DHH // 16, _zero_num, 0)

    def _zero_den(i, _):
        plsc.store_scatter(den_acc, [i * 16 + lane], zero16)
        return 0

    lax.fori_loop(0, N // 16, _zero_den, 0)

    def _block(blk, _):
        gblk = eh * BLKS_PER_HALF + blk
        d1 = pltpu.async_copy(dst_hbm.at[pl.ds(gblk * B, B)], idx_dst, sem)
        d2 = pltpu.async_copy(ex_hbm.at[pl.ds(gblk * H * B + ht * B, B)],
                              ex_s, sem)
        d3 = pltpu.async_copy(
            msg_hbm.at[pl.ds(gblk * MB + (ht * DH + dhh * DHH) * B,
                             DHH * B)], msg_s, sem)
        d1.wait()
        d2.wait()
        d3.wait()

        def _group(g, _):
            rows = g * 16 + lane
            dstv = plsc.load_gather(idx_dst, [rows])
            exv = plsc.load_gather(ex_s, [rows])
            plsc.addupdate_scatter(den_acc, [dstv], exv, mask=den_mask)
            for d in range(DHH):
                mv = plsc.load_gather(msg_s, [d * B + rows])
                plsc.addupdate_scatter(num_acc, [dstv * DHH + d], mv)
            return 0

        lax.fori_loop(0, B // 16, _group, 0)
        return 0

    lax.fori_loop(0, BLKS_PER_HALF, _block, 0)

    pltpu.sync_copy(num_acc, num_out.at[pl.ds(wid * N * DHH, N * DHH)])
    pltpu.sync_copy(den_acc, den_out.at[pl.ds(wid * N, N)])


def _sc_agg_call(dst, ex, msg):
    mesh = plsc.VectorSubcoreMesh(core_axis_name="c", subcore_axis_name="s")
    f = pl.kernel(
        _sc_agg_body,
        out_type=[
            jax.ShapeDtypeStruct((NUM_TILES * N * DHH,), jnp.float32),
            jax.ShapeDtypeStruct((NUM_TILES * N,), jnp.float32),
        ],
        mesh=mesh,
        scratch_types=[
            pltpu.VMEM((B,), jnp.int32),
            pltpu.VMEM((B,), jnp.float32),
            pltpu.VMEM((DHH * B,), jnp.float32),
            pltpu.VMEM((N * DHH,), jnp.float32),
            pltpu.VMEM((N,), jnp.float32),
            pltpu.SemaphoreType.DMA,
        ],
        compiler_params=pltpu.CompilerParams(needs_layout_passes=False),
    )
    return f(dst, ex, msg)


def _final_body(*refs):
    num_refs = refs[0:32]
    den_refs = refs[32:48]
    x_ref, wo_ref, lns_ref, lnb_ref, out_ref = refs[48:]
    agg_cols = []
    den_cols = []
    for h in range(H):
        for dhh in range(2):
            i0 = h * 4 + dhh * 2
            agg_cols.append(num_refs[i0][0] + num_refs[i0 + 1][0])
        dsum = den_refs[2 * h][0] + den_refs[2 * h + 1][0]
        den_cols.append(jnp.broadcast_to(dsum, (dsum.shape[0], DH)))
    agg = jnp.concatenate(agg_cols, axis=1)
    dfull = jnp.concatenate(den_cols, axis=1)
    aggn = agg / (dfull + 1e-9)
    out = jnp.dot(aggn, wo_ref[...], preferred_element_type=jnp.float32)
    hres = x_ref[...] + out
    mu = jnp.mean(hres, axis=1, keepdims=True)
    var = jnp.mean((hres - mu) ** 2, axis=1, keepdims=True)
    out_ref[...] = ((hres - mu) / jnp.sqrt(var + 1e-5)) * lns_ref[...] \
        + lnb_ref[...]


def _final_call(num, den, x, wo, lns, lnb):
    bn = 1000

    def _num_spec(tile):
        return pl.BlockSpec((1, bn, DHH), lambda i, _t=tile: (_t, i, 0))

    def _den_spec(tile):
        return pl.BlockSpec((1, bn, 1), lambda i, _t=tile: (_t, i, 0))

    in_specs = [_num_spec(h * 4 + dhh * 2 + e)
                for h in range(H) for dhh in range(2) for e in range(2)]
    in_specs += [_den_spec(h * 4 + e) for h in range(H) for e in range(2)]
    in_specs += [
        pl.BlockSpec((bn, D), lambda i: (i, 0)),
        pl.BlockSpec((D, D), lambda i: (0, 0)),
        pl.BlockSpec((1, D), lambda i: (0, 0)),
        pl.BlockSpec((1, D), lambda i: (0, 0)),
    ]
    num3 = num.reshape(NUM_TILES, N, DHH)
    den3 = den.reshape(NUM_TILES, N, 1)
    args = [num3] * 32 + [den3] * 16 + [x, wo, lns, lnb]
    return pl.pallas_call(
        _final_body,
        grid=(N // bn,),
        in_specs=in_specs,
        out_specs=pl.BlockSpec((bn, D), lambda i: (i, 0)),
        out_shape=jax.ShapeDtypeStruct((N, D), jnp.float32),
    )(*args)


def kernel(x, edge_index, Wq, Wk, Wv, Wo, ln_scale, ln_bias):
    q, k, v = _qkv_call(x, Wq, Wk, Wv)
    src = edge_index[0]
    dst = edge_index[1]
    msg, ex = _sc_msg_call(q, k, v, src, dst)
    num, den = _sc_agg_call(dst, ex, msg)
    out = _final_call(num, den, x, Wo,
                      ln_scale.reshape(1, D), ln_bias.reshape(1, D))
    return out


# speedup vs baseline: 10.7359x; 1.1778x over previous
import jax
import jax.numpy as jnp
from jax import lax
from jax.experimental import pallas as pl
from jax.experimental.pallas import tpu as pltpu
from jax.experimental.pallas import tpu_sc as plsc

N = 10000
E = 320000
D = 128
H = 8
DH = 16

NUM_CORES = 2
NUM_SUBCORES = 16
NUM_TILES = NUM_CORES * NUM_SUBCORES
B = 80
NBLK = E // B
BLKS_PER_TILE = NBLK // NUM_TILES
BLKS_PER_HALF = NBLK // 2
MB = H * DH * B
DHH = DH // 2


def _qkv_body(x_ref, wq_ref, wk_ref, wv_ref, q_ref, k_ref, v_ref):
    xb = x_ref[...]
    q_ref[...] = jnp.dot(xb, wq_ref[...], preferred_element_type=jnp.float32)
    k_ref[...] = jnp.dot(xb, wk_ref[...], preferred_element_type=jnp.float32)
    v_ref[...] = jnp.dot(xb, wv_ref[...], preferred_element_type=jnp.float32)


def _qkv_call(x, wq, wk, wv):
    bn = 1000
    w_spec = pl.BlockSpec((D, D), lambda i: (0, 0))
    r_spec = pl.BlockSpec((bn, D), lambda i: (i, 0))
    return pl.pallas_call(
        _qkv_body,
        grid=(N // bn,),
        in_specs=[r_spec, w_spec, w_spec, w_spec],
        out_specs=[r_spec, r_spec, r_spec],
        out_shape=[
            jax.ShapeDtypeStruct((N, D), jnp.float32),
            jax.ShapeDtypeStruct((N, D), jnp.float32),
            jax.ShapeDtypeStruct((N, D), jnp.float32),
        ],
    )(x, wq, wk, wv)


def _sc_msg_body(q_hbm, k_hbm, v_hbm, src_hbm, dst_hbm, msg_out, ex_out,
                 idx_src, idx_dst, q_buf, kv_buf, msg_buf, ex_buf, sem):
    c = lax.axis_index("c")
    s = lax.axis_index("s")
    wid = s * NUM_CORES + c
    lane = jnp.arange(16, dtype=jnp.int32)
    zero16 = jnp.zeros((16,), jnp.float32)

    def _block(blk, _):
        gblk = wid * BLKS_PER_TILE + blk
        base = gblk * B
        d1 = pltpu.async_copy(src_hbm.at[pl.ds(base, B)], idx_src, sem)
        d2 = pltpu.async_copy(dst_hbm.at[pl.ds(base, B)], idx_dst, sem)
        d1.wait()
        d2.wait()
        d3 = pltpu.async_copy(q_hbm.at[idx_dst], q_buf, sem)
        d4 = pltpu.async_copy(k_hbm.at[idx_src], kv_buf, sem)
        d3.wait()
        d4.wait()

        def _score_group(g, _):
            rows = g * 16 + lane

            def _score_head(h, _):
                acc = zero16
                for d in range(DH):
                    col = jnp.full((16,), h * DH + d, jnp.int32)
                    qv = plsc.load_gather(q_buf, [rows, col])
                    kv = plsc.load_gather(kv_buf, [rows, col])
                    acc = acc + qv * kv
                ex = jnp.exp(acc * 0.25)
                plsc.store_scatter(ex_buf, [h * B + rows], ex)
                return 0

            return lax.fori_loop(0, H, _score_head, 0)

        lax.fori_loop(0, B // 16, _score_group, 0)

        pltpu.sync_copy(v_hbm.at[idx_src], kv_buf)

        def _msg_group(g, _):
            rows = g * 16 + lane

            def _msg_head(h, _):
                exv = plsc.load_gather(ex_buf, [h * B + rows])
                for d in range(DH):
                    colv = jnp.full((16,), h * DH + d, jnp.int32)
                    vv = plsc.load_gather(kv_buf, [rows, colv])
                    plsc.store_scatter(msg_buf,
                                       [(h * DH + d) * B + rows], vv * exv)
                return 0

            return lax.fori_loop(0, H, _msg_head, 0)

        lax.fori_loop(0, B // 16, _msg_group, 0)
        d5 = pltpu.async_copy(msg_buf, msg_out.at[pl.ds(gblk * MB, MB)], sem)
        d6 = pltpu.async_copy(ex_buf, ex_out.at[pl.ds(gblk * H * B, H * B)],
                              sem)
        d5.wait()
        d6.wait()
        return 0

    lax.fori_loop(0, BLKS_PER_TILE, _block, 0)


def _sc_msg_call(q, k, v, src, dst):
    mesh = plsc.VectorSubcoreMesh(core_axis_name="c", subcore_axis_name="s")
    f = pl.kernel(
        _sc_msg_body,
        out_type=[
            jax.ShapeDtypeStruct((NBLK * MB,), jnp.float32),
            jax.ShapeDtypeStruct((NBLK * H * B,), jnp.float32),
        ],
        mesh=mesh,
        scratch_types=[
            pltpu.VMEM((B,), jnp.int32),
            pltpu.VMEM((B,), jnp.int32),
            pltpu.VMEM((B, D), jnp.float32),
            pltpu.VMEM((B, D), jnp.float32),
            pltpu.VMEM((MB,), jnp.float32),
            pltpu.VMEM((H * B,), jnp.float32),
            pltpu.SemaphoreType.DMA,
        ],
        compiler_params=pltpu.CompilerParams(needs_layout_passes=False),
    )
    return f(q, k, v, src, dst)


def _sc_agg_body(dst_hbm, ex_hbm, msg_hbm, num_out, den_out,
                 idx_dst, ex_s, msg_s, num_acc, den_acc, sem):
    c = lax.axis_index("c")
    s = lax.axis_index("s")
    wid = s * NUM_CORES + c
    ht = wid // 4
    dhh = (wid // 2) % 2
    eh = wid % 2
    lane = jnp.arange(16, dtype=jnp.int32)
    zero16 = jnp.zeros((16,), jnp.float32)
    den_mask = jnp.full((16,), dhh == 0)

    def _zero_num(i, _):
        plsc.store_scatter(num_acc, [i * 16 + lane], zero16)
        return 0

    lax.fori_loop(0, N * DHH // 16, _zero_num, 0)

    def _zero_den(i, _):
        plsc.store_scatter(den_acc, [i * 16 + lane], zero16)
        return 0

    lax.fori_loop(0, N // 16, _zero_den, 0)

    QUAD = 4

    def _block(it, _):
        gblk0 = eh * BLKS_PER_HALF + it * QUAD
        ds_list = []
        for j in range(QUAD):
            gblk = gblk0 + j
            ds_list.append(pltpu.async_copy(
                dst_hbm.at[pl.ds(gblk * B, B)],
                idx_dst.at[pl.ds(j * B, B)], sem))
            ds_list.append(pltpu.async_copy(
                ex_hbm.at[pl.ds(gblk * H * B + ht * B, B)],
                ex_s.at[pl.ds(j * B, B)], sem))
            ds_list.append(pltpu.async_copy(
                msg_hbm.at[pl.ds(gblk * MB + (ht * DH + dhh * DHH) * B,
                                 DHH * B)],
                msg_s.at[pl.ds(j * DHH * B, DHH * B)], sem))
        for dsc in ds_list:
            dsc.wait()

        for j in range(QUAD):
            def _group(g, _, _j=j):
                rows = g * 16 + lane
                dstv = plsc.load_gather(idx_dst, [_j * B + rows])
                exv = plsc.load_gather(ex_s, [_j * B + rows])
                plsc.addupdate_scatter(den_acc, [dstv], exv, mask=den_mask)
                for d in range(DHH):
                    mv = plsc.load_gather(msg_s, [(_j * DHH + d) * B + rows])
                    plsc.addupdate_scatter(num_acc, [dstv * DHH + d], mv)
                return 0

            lax.fori_loop(0, B // 16, _group, 0)
        return 0

    lax.fori_loop(0, BLKS_PER_HALF // QUAD, _block, 0)

    pltpu.sync_copy(num_acc, num_out.at[pl.ds(wid * N * DHH, N * DHH)])
    pltpu.sync_copy(den_acc, den_out.at[pl.ds(wid * N, N)])


def _sc_agg_call(dst, ex, msg):
    mesh = plsc.VectorSubcoreMesh(core_axis_name="c", subcore_axis_name="s")
    f = pl.kernel(
        _sc_agg_body,
        out_type=[
            jax.ShapeDtypeStruct((NUM_TILES * N * DHH,), jnp.float32),
            jax.ShapeDtypeStruct((NUM_TILES * N,), jnp.float32),
        ],
        mesh=mesh,
        scratch_types=[
            pltpu.VMEM((4 * B,), jnp.int32),
            pltpu.VMEM((4 * B,), jnp.float32),
            pltpu.VMEM((4 * DHH * B,), jnp.float32),
            pltpu.VMEM((N * DHH,), jnp.float32),
            pltpu.VMEM((N,), jnp.float32),
            pltpu.SemaphoreType.DMA,
        ],
        compiler_params=pltpu.CompilerParams(needs_layout_passes=False),
    )
    return f(dst, ex, msg)


def _final_body(*refs):
    num_refs = refs[0:32]
    den_refs = refs[32:48]
    x_ref, wo_ref, lns_ref, lnb_ref, out_ref = refs[48:]
    agg_cols = []
    den_cols = []
    for h in range(H):
        for dhh in range(2):
            i0 = h * 4 + dhh * 2
            agg_cols.append(num_refs[i0][0] + num_refs[i0 + 1][0])
        dsum = den_refs[2 * h][0] + den_refs[2 * h + 1][0]
        den_cols.append(jnp.broadcast_to(dsum, (dsum.shape[0], DH)))
    agg = jnp.concatenate(agg_cols, axis=1)
    dfull = jnp.concatenate(den_cols, axis=1)
    aggn = agg / (dfull + 1e-9)
    out = jnp.dot(aggn, wo_ref[...], preferred_element_type=jnp.float32)
    hres = x_ref[...] + out
    mu = jnp.mean(hres, axis=1, keepdims=True)
    var = jnp.mean((hres - mu) ** 2, axis=1, keepdims=True)
    out_ref[...] = ((hres - mu) / jnp.sqrt(var + 1e-5)) * lns_ref[...] \
        + lnb_ref[...]


def _final_call(num, den, x, wo, lns, lnb):
    bn = 1000

    def _num_spec(tile):
        return pl.BlockSpec((1, bn, DHH), lambda i, _t=tile: (_t, i, 0))

    def _den_spec(tile):
        return pl.BlockSpec((1, bn, 1), lambda i, _t=tile: (_t, i, 0))

    in_specs = [_num_spec(h * 4 + dhh * 2 + e)
                for h in range(H) for dhh in range(2) for e in range(2)]
    in_specs += [_den_spec(h * 4 + e) for h in range(H) for e in range(2)]
    in_specs += [
        pl.BlockSpec((bn, D), lambda i: (i, 0)),
        pl.BlockSpec((D, D), lambda i: (0, 0)),
        pl.BlockSpec((1, D), lambda i: (0, 0)),
        pl.BlockSpec((1, D), lambda i: (0, 0)),
    ]
    num3 = num.reshape(NUM_TILES, N, DHH)
    den3 = den.reshape(NUM_TILES, N, 1)
    args = [num3] * 32 + [den3] * 16 + [x, wo, lns, lnb]
    return pl.pallas_call(
        _final_body,
        grid=(N // bn,),
        in_specs=in_specs,
        out_specs=pl.BlockSpec((bn, D), lambda i: (i, 0)),
        out_shape=jax.ShapeDtypeStruct((N, D), jnp.float32),
    )(*args)


def kernel(x, edge_index, Wq, Wk, Wv, Wo, ln_scale, ln_bias):
    q, k, v = _qkv_call(x, Wq, Wk, Wv)
    src = edge_index[0]
    dst = edge_index[1]
    msg, ex = _sc_msg_call(q, k, v, src, dst)
    num, den = _sc_agg_call(dst, ex, msg)
    out = _final_call(num, den, x, Wo,
                      ln_scale.reshape(1, D), ln_bias.reshape(1, D))
    return out


# speedup vs baseline: 10.9469x; 1.0197x over previous
import jax
import jax.numpy as jnp
from jax import lax
from jax.experimental import pallas as pl
from jax.experimental.pallas import tpu as pltpu
from jax.experimental.pallas import tpu_sc as plsc

N = 10000
E = 320000
D = 128
H = 8
DH = 16

NUM_CORES = 2
NUM_SUBCORES = 16
NUM_TILES = NUM_CORES * NUM_SUBCORES
B = 80
NBLK = E // B
BLKS_PER_TILE = NBLK // NUM_TILES
BLKS_PER_HALF = NBLK // 2
MB = H * DH * B
DHH = DH // 2


def _qkv_body(x_ref, wq_ref, wk_ref, wv_ref, q_ref, k_ref, v_ref):
    xb = x_ref[...]
    q_ref[...] = jnp.dot(xb, wq_ref[...], preferred_element_type=jnp.float32)
    k_ref[...] = jnp.dot(xb, wk_ref[...], preferred_element_type=jnp.float32)
    v_ref[...] = jnp.dot(xb, wv_ref[...], preferred_element_type=jnp.float32)


def _qkv_call(x, wq, wk, wv):
    bn = 1000
    w_spec = pl.BlockSpec((D, D), lambda i: (0, 0))
    r_spec = pl.BlockSpec((bn, D), lambda i: (i, 0))
    return pl.pallas_call(
        _qkv_body,
        grid=(N // bn,),
        in_specs=[r_spec, w_spec, w_spec, w_spec],
        out_specs=[r_spec, r_spec, r_spec],
        out_shape=[
            jax.ShapeDtypeStruct((N, D), jnp.float32),
            jax.ShapeDtypeStruct((N, D), jnp.float32),
            jax.ShapeDtypeStruct((N, D), jnp.float32),
        ],
    )(x, wq, wk, wv)


def _sc_msg_body(q_hbm, k_hbm, v_hbm, src_hbm, dst_hbm, msg_out, ex_out,
                 idx_src, idx_dst, q_buf, kv_buf, v_buf, msg_buf, ex_buf,
                 sem):
    c = lax.axis_index("c")
    s = lax.axis_index("s")
    wid = s * NUM_CORES + c
    lane = jnp.arange(16, dtype=jnp.int32)
    zero16 = jnp.zeros((16,), jnp.float32)

    def _block(blk, _):
        gblk = wid * BLKS_PER_TILE + blk
        base = gblk * B
        d1 = pltpu.async_copy(src_hbm.at[pl.ds(base, B)], idx_src, sem)
        d2 = pltpu.async_copy(dst_hbm.at[pl.ds(base, B)], idx_dst, sem)
        d1.wait()
        d2.wait()
        d3 = pltpu.async_copy(q_hbm.at[idx_dst], q_buf, sem)
        d4 = pltpu.async_copy(k_hbm.at[idx_src], kv_buf, sem)
        d5 = pltpu.async_copy(v_hbm.at[idx_src], v_buf, sem)
        d3.wait()
        d4.wait()
        d5.wait()

        def _score_group(g, _):
            rows = g * 16 + lane

            def _score_head(h, _):
                acc = zero16
                for d in range(DH):
                    col = jnp.full((16,), h * DH + d, jnp.int32)
                    qv = plsc.load_gather(q_buf, [rows, col])
                    kv = plsc.load_gather(kv_buf, [rows, col])
                    acc = acc + qv * kv
                ex = jnp.exp(acc * 0.25)
                plsc.store_scatter(ex_buf, [h * B + rows], ex)
                return 0

            return lax.fori_loop(0, H, _score_head, 0)

        lax.fori_loop(0, B // 16, _score_group, 0)

        def _msg_group(g, _):
            rows = g * 16 + lane

            def _msg_head(h, _):
                exv = plsc.load_gather(ex_buf, [h * B + rows])
                for d in range(DH):
                    colv = jnp.full((16,), h * DH + d, jnp.int32)
                    vv = plsc.load_gather(v_buf, [rows, colv])
                    plsc.store_scatter(msg_buf,
                                       [(h * DH + d) * B + rows], vv * exv)
                return 0

            return lax.fori_loop(0, H, _msg_head, 0)

        lax.fori_loop(0, B // 16, _msg_group, 0)
        d6 = pltpu.async_copy(msg_buf, msg_out.at[pl.ds(gblk * MB, MB)], sem)
        d7 = pltpu.async_copy(ex_buf, ex_out.at[pl.ds(gblk * H * B, H * B)],
                              sem)
        d6.wait()
        d7.wait()
        return 0

    lax.fori_loop(0, BLKS_PER_TILE, _block, 0)


def _sc_msg_call(q, k, v, src, dst):
    mesh = plsc.VectorSubcoreMesh(core_axis_name="c", subcore_axis_name="s")
    f = pl.kernel(
        _sc_msg_body,
        out_type=[
            jax.ShapeDtypeStruct((NBLK * MB,), jnp.float32),
            jax.ShapeDtypeStruct((NBLK * H * B,), jnp.float32),
        ],
        mesh=mesh,
        scratch_types=[
            pltpu.VMEM((B,), jnp.int32),
            pltpu.VMEM((B,), jnp.int32),
            pltpu.VMEM((B, D), jnp.float32),
            pltpu.VMEM((B, D), jnp.float32),
            pltpu.VMEM((B, D), jnp.float32),
            pltpu.VMEM((MB,), jnp.float32),
            pltpu.VMEM((H * B,), jnp.float32),
            pltpu.SemaphoreType.DMA,
        ],
        compiler_params=pltpu.CompilerParams(needs_layout_passes=False),
    )
    return f(q, k, v, src, dst)


def _sc_agg_body(dst_hbm, ex_hbm, msg_hbm, num_out, den_out,
                 idx_dst, ex_s, msg_s, num_acc, den_acc, sem):
    c = lax.axis_index("c")
    s = lax.axis_index("s")
    wid = s * NUM_CORES + c
    ht = wid // 4
    dhh = (wid // 2) % 2
    eh = wid % 2
    lane = jnp.arange(16, dtype=jnp.int32)
    zero16 = jnp.zeros((16,), jnp.float32)
    den_mask = jnp.full((16,), dhh == 0)

    def _zero_num(i, _):
        plsc.store_scatter(num_acc, [i * 16 + lane], zero16)
        return 0

    lax.fori_loop(0, N * DHH // 16, _zero_num, 0)

    def _zero_den(i, _):
        plsc.store_scatter(den_acc, [i * 16 + lane], zero16)
        return 0

    lax.fori_loop(0, N // 16, _zero_den, 0)

    QUAD = 4

    def _block(it, _):
        gblk0 = eh * BLKS_PER_HALF + it * QUAD
        ds_list = []
        for j in range(QUAD):
            gblk = gblk0 + j
            ds_list.append(pltpu.async_copy(
                dst_hbm.at[pl.ds(gblk * B, B)],
                idx_dst.at[pl.ds(j * B, B)], sem))
            ds_list.append(pltpu.async_copy(
                ex_hbm.at[pl.ds(gblk * H * B + ht * B, B)],
                ex_s.at[pl.ds(j * B, B)], sem))
            ds_list.append(pltpu.async_copy(
                msg_hbm.at[pl.ds(gblk * MB + (ht * DH + dhh * DHH) * B,
                                 DHH * B)],
                msg_s.at[pl.ds(j * DHH * B, DHH * B)], sem))
        for dsc in ds_list:
            dsc.wait()

        for j in range(QUAD):
            def _group(g, _, _j=j):
                rows = g * 16 + lane
                dstv = plsc.load_gather(idx_dst, [_j * B + rows])
                exv = plsc.load_gather(ex_s, [_j * B + rows])
                plsc.addupdate_scatter(den_acc, [dstv], exv, mask=den_mask)
                for d in range(DHH):
                    mv = plsc.load_gather(msg_s, [(_j * DHH + d) * B + rows])
                    plsc.addupdate_scatter(num_acc, [dstv * DHH + d], mv)
                return 0

            lax.fori_loop(0, B // 16, _group, 0)
        return 0

    lax.fori_loop(0, BLKS_PER_HALF // QUAD, _block, 0)

    pltpu.sync_copy(num_acc, num_out.at[pl.ds(wid * N * DHH, N * DHH)])
    pltpu.sync_copy(den_acc, den_out.at[pl.ds(wid * N, N)])


def _sc_agg_call(dst, ex, msg):
    mesh = plsc.VectorSubcoreMesh(core_axis_name="c", subcore_axis_name="s")
    f = pl.kernel(
        _sc_agg_body,
        out_type=[
            jax.ShapeDtypeStruct((NUM_TILES * N * DHH,), jnp.float32),
            jax.ShapeDtypeStruct((NUM_TILES * N,), jnp.float32),
        ],
        mesh=mesh,
        scratch_types=[
            pltpu.VMEM((4 * B,), jnp.int32),
            pltpu.VMEM((4 * B,), jnp.float32),
            pltpu.VMEM((4 * DHH * B,), jnp.float32),
            pltpu.VMEM((N * DHH,), jnp.float32),
            pltpu.VMEM((N,), jnp.float32),
            pltpu.SemaphoreType.DMA,
        ],
        compiler_params=pltpu.CompilerParams(needs_layout_passes=False),
    )
    return f(dst, ex, msg)


def _final_body(*refs):
    num_refs = refs[0:32]
    den_refs = refs[32:48]
    x_ref, wo_ref, lns_ref, lnb_ref, out_ref = refs[48:]
    agg_cols = []
    den_cols = []
    for h in range(H):
        for dhh in range(2):
            i0 = h * 4 + dhh * 2
            agg_cols.append(num_refs[i0][0] + num_refs[i0 + 1][0])
        dsum = den_refs[2 * h][0] + den_refs[2 * h + 1][0]
        den_cols.append(jnp.broadcast_to(dsum, (dsum.shape[0], DH)))
    agg = jnp.concatenate(agg_cols, axis=1)
    dfull = jnp.concatenate(den_cols, axis=1)
    aggn = agg / (dfull + 1e-9)
    out = jnp.dot(aggn, wo_ref[...], preferred_element_type=jnp.float32)
    hres = x_ref[...] + out
    mu = jnp.mean(hres, axis=1, keepdims=True)
    var = jnp.mean((hres - mu) ** 2, axis=1, keepdims=True)
    out_ref[...] = ((hres - mu) / jnp.sqrt(var + 1e-5)) * lns_ref[...] \
        + lnb_ref[...]


def _final_call(num, den, x, wo, lns, lnb):
    bn = 1000

    def _num_spec(tile):
        return pl.BlockSpec((1, bn, DHH), lambda i, _t=tile: (_t, i, 0))

    def _den_spec(tile):
        return pl.BlockSpec((1, bn, 1), lambda i, _t=tile: (_t, i, 0))

    in_specs = [_num_spec(h * 4 + dhh * 2 + e)
                for h in range(H) for dhh in range(2) for e in range(2)]
    in_specs += [_den_spec(h * 4 + e) for h in range(H) for e in range(2)]
    in_specs += [
        pl.BlockSpec((bn, D), lambda i: (i, 0)),
        pl.BlockSpec((D, D), lambda i: (0, 0)),
        pl.BlockSpec((1, D), lambda i: (0, 0)),
        pl.BlockSpec((1, D), lambda i: (0, 0)),
    ]
    num3 = num.reshape(NUM_TILES, N, DHH)
    den3 = den.reshape(NUM_TILES, N, 1)
    args = [num3] * 32 + [den3] * 16 + [x, wo, lns, lnb]
    return pl.pallas_call(
        _final_body,
        grid=(N // bn,),
        in_specs=in_specs,
        out_specs=pl.BlockSpec((bn, D), lambda i: (i, 0)),
        out_shape=jax.ShapeDtypeStruct((N, D), jnp.float32),
    )(*args)


def kernel(x, edge_index, Wq, Wk, Wv, Wo, ln_scale, ln_bias):
    q, k, v = _qkv_call(x, Wq, Wk, Wv)
    src = edge_index[0]
    dst = edge_index[1]
    msg, ex = _sc_msg_call(q, k, v, src, dst)
    num, den = _sc_agg_call(dst, ex, msg)
    out = _final_call(num, den, x, Wo,
                      ln_scale.reshape(1, D), ln_bias.reshape(1, D))
    return out
